# baseline (speedup 1.0000x reference)
import functools

import jax
import jax.numpy as jnp
from jax import lax
from jax.experimental import pallas as pl
from jax.experimental.pallas import tpu as pltpu
from jax.experimental.pallas import tpu_sc as plsc

D_MODEL = 2048
NUM_TOKENS = 4 * 8192
NC, NS = 2, 16
NW = NC * NS
B_PER_W = NUM_TOKENS // NW
CHUNK = 32
N_CHUNKS = B_PER_W // CHUNK


def _lookup_body(idx_hbm, table_hbm, out_hbm, idx_v, buf, sem):
    wid = lax.axis_index("s") * NC + lax.axis_index("c")
    base = wid * B_PER_W
    pltpu.sync_copy(idx_hbm.at[pl.ds(base, B_PER_W)], idx_v)

    def chunk(j, carry):
        off = j * CHUNK
        pltpu.async_copy(
            table_hbm.at[idx_v.at[pl.ds(off, CHUNK)]], buf, sem
        ).wait()
        pltpu.sync_copy(buf, out_hbm.at[pl.ds(base + off, CHUNK)])
        return carry

    lax.fori_loop(0, N_CHUNKS, chunk, 0)


_lookup = functools.partial(
    pl.kernel,
    out_type=jax.ShapeDtypeStruct((NUM_TOKENS, D_MODEL), jnp.float32),
    mesh=plsc.VectorSubcoreMesh(core_axis_name="c", subcore_axis_name="s"),
    scratch_types=[
        pltpu.VMEM((B_PER_W,), jnp.int32),
        pltpu.VMEM((CHUNK, D_MODEL), jnp.float32),
        pltpu.SemaphoreType.DMA,
    ],
)(_lookup_body)


def kernel(modality_indices, table):
    idx = modality_indices.reshape(-1).astype(jnp.int32)
    out = _lookup(idx, table)
    return out.reshape(*modality_indices.shape, table.shape[1])


# speedup vs baseline: 1.0015x; 1.0015x over previous
import functools

import jax
import jax.numpy as jnp
from jax import lax
from jax.experimental import pallas as pl
from jax.experimental.pallas import tpu as pltpu
from jax.experimental.pallas import tpu_sc as plsc

D_MODEL = 2048
NUM_TOKENS = 4 * 8192
NC, NS = 2, 16
NW = NC * NS
B_PER_W = NUM_TOKENS // NW
CHUNK = 32
N_CHUNKS = B_PER_W // CHUNK


def _lookup_body(idx_hbm, table_hbm, out_hbm, idx_v, buf, sem):
    sid = lax.axis_index("s")
    wid = sid * NC + lax.axis_index("c")
    base = wid * B_PER_W
    pltpu.sync_copy(idx_hbm.at[pl.ds(base, B_PER_W)], idx_v)

    def chunk(j, carry):
        off = j * CHUNK
        pltpu.async_copy(
            table_hbm.at[idx_v.at[pl.ds(off, CHUNK)]], buf, sem
        ).wait()
        pltpu.sync_copy(buf, out_hbm.at[pl.ds(base + off, CHUNK)])
        return carry

    lax.fori_loop(0, N_CHUNKS, chunk, 0)


_lookup = functools.partial(
    pl.kernel,
    out_type=jax.ShapeDtypeStruct((NUM_TOKENS, D_MODEL), jnp.float32),
    mesh=plsc.VectorSubcoreMesh(core_axis_name="c", subcore_axis_name="s"),
    scratch_types=[
        pltpu.VMEM((B_PER_W,), jnp.int32),
        pltpu.VMEM((CHUNK, D_MODEL), jnp.float32),
        pltpu.SemaphoreType.DMA,
    ],
)(_lookup_body)


def kernel(modality_indices, table):
    idx = modality_indices.reshape(-1).astype(jnp.int32)
    out = _lookup(idx, table)
    return out.reshape(*modality_indices.shape, table.shape[1])


# speedup vs baseline: 6.2500x; 6.2408x over previous
import functools

import jax
import jax.numpy as jnp
from jax import lax
from jax.experimental import pallas as pl
from jax.experimental.pallas import tpu as pltpu
from jax.experimental.pallas import tpu_sc as plsc

NUM_MOD = 8
D_MODEL = 2048
NUM_TOKENS = 4 * 8192
NC, NS = 2, 16
NW = NC * NS
B_PER_W = NUM_TOKENS // NW
GROUP = 16
N_GROUPS = B_PER_W // GROUP


def _lookup_body(idx_hbm, table_hbm, out_hbm, idx_v, table_v, sem):
    sid = lax.axis_index("s")
    wid = sid * NC + lax.axis_index("c")
    base = wid * B_PER_W
    pltpu.sync_copy(idx_hbm.at[pl.ds(base, B_PER_W)], idx_v)
    pltpu.sync_copy(table_hbm, table_v)

    def group(g, carry):
        off = g * GROUP
        vec = idx_v[pl.ds(off, GROUP)]
        for u in range(GROUP):
            i = vec[u]
            pltpu.async_copy(
                table_v.at[pl.ds(i, 1)], out_hbm.at[pl.ds(base + off + u, 1)], sem
            )
        for u in range(GROUP):
            i = vec[u]
            pltpu.make_async_copy(
                table_v.at[pl.ds(i, 1)], out_hbm.at[pl.ds(base + off + u, 1)], sem
            ).wait()
        return carry

    lax.fori_loop(0, N_GROUPS, group, 0)


_lookup = functools.partial(
    pl.kernel,
    out_type=jax.ShapeDtypeStruct((NUM_TOKENS, D_MODEL), jnp.float32),
    mesh=plsc.VectorSubcoreMesh(core_axis_name="c", subcore_axis_name="s"),
    scratch_types=[
        pltpu.VMEM((B_PER_W,), jnp.int32),
        pltpu.VMEM((NUM_MOD, D_MODEL), jnp.float32),
        pltpu.SemaphoreType.DMA,
    ],
)(_lookup_body)


def kernel(modality_indices, table):
    idx = modality_indices.reshape(-1).astype(jnp.int32)
    out = _lookup(idx, table)
    return out.reshape(*modality_indices.shape, table.shape[1])
